# baseline (device time: 13206 ns/iter reference)
import jax
import jax.numpy as jnp
from jax import lax
from jax.experimental import pallas as pl
from jax.experimental.pallas import tpu as pltpu

N_Z = 4
CHUNK = 256

_TOPO = {
    0: ([1], 2, 3),
    1: ([0, 2], 3, None),
    2: ([1, 3], 0, None),
    3: ([2], 1, 0),
}


def kernel(x):
    _, m, n = x.shape
    assert (m, n) == (256, N_Z * CHUNK)

    def body(x_ref, out_ref, send_ref, recv_ref, send_sems, recv_sems,
             d2_sem, d3_sem):
        my_x = lax.axis_index("x")
        my_y = lax.axis_index("y")
        my_z = lax.axis_index("z")
        near_sem = pltpu.get_barrier_semaphore()

        def dev(z):
            return (my_x, my_y, z)

        def emit(z):
            nbrs, d2p, d3p = _TOPO[z]

            for nb in nbrs:
                pl.semaphore_signal(near_sem, inc=1, device_id=dev(nb),
                                    device_id_type=pl.DeviceIdType.MESH)
            pl.semaphore_signal(d2_sem, inc=1, device_id=dev(d2p),
                                device_id_type=pl.DeviceIdType.MESH)
            if d3p is not None:
                pl.semaphore_signal(d3_sem, inc=1, device_id=dev(d3p),
                                    device_id_type=pl.DeviceIdType.MESH)

            dests = nbrs + [d2p] + ([d3p] if d3p is not None else [])
            for d in dests:
                send_ref[d] = x_ref[0, :, d * CHUNK:(d + 1) * CHUNK].astype(
                    jnp.bfloat16
                )

            def start(d):
                rdma = pltpu.make_async_remote_copy(
                    src_ref=send_ref.at[d],
                    dst_ref=recv_ref.at[z],
                    send_sem=send_sems.at[d],
                    recv_sem=recv_sems.at[z],
                    device_id=dev(d),
                    device_id_type=pl.DeviceIdType.MESH,
                )
                rdma.start()
                return rdma

            rdmas = []
            pl.semaphore_wait(near_sem, len(nbrs))
            rdmas += [start(d) for d in nbrs]
            pl.semaphore_wait(d2_sem, 1)
            rdmas.append(start(d2p))
            if d3p is not None:
                pl.semaphore_wait(d3_sem, 1)
                rdmas.append(start(d3p))

            acc = x_ref[0, :, z * CHUNK:(z + 1) * CHUNK]
            for s in sorted(dests, key=lambda s: abs(s - z)):
                recv = pltpu.make_async_remote_copy(
                    src_ref=send_ref.at[s],
                    dst_ref=recv_ref.at[s],
                    send_sem=send_sems.at[s],
                    recv_sem=recv_sems.at[s],
                    device_id=dev(s),
                    device_id_type=pl.DeviceIdType.MESH,
                )
                recv.wait_recv()
                acc = acc + recv_ref[s].astype(jnp.float32)
            out_ref[...] = acc

            for rdma in rdmas:
                rdma.wait_send()

        for z in range(N_Z):
            pl.when(my_z == z)(lambda z=z: emit(z))

    return pl.pallas_call(
        body,
        out_shape=jax.ShapeDtypeStruct((m, CHUNK), jnp.float32),
        in_specs=[pl.BlockSpec(memory_space=pltpu.VMEM)],
        out_specs=pl.BlockSpec(memory_space=pltpu.VMEM),
        scratch_shapes=[
            pltpu.VMEM((N_Z, m, CHUNK), jnp.bfloat16),
            pltpu.VMEM((N_Z, m, CHUNK), jnp.bfloat16),
            pltpu.SemaphoreType.DMA((N_Z,)),
            pltpu.SemaphoreType.DMA((N_Z,)),
            pltpu.SemaphoreType.REGULAR,
            pltpu.SemaphoreType.REGULAR,
        ],
        compiler_params=pltpu.CompilerParams(collective_id=0),
    )(x)
